# baseline (speedup 1.0000x reference)
import jax
import jax.numpy as jnp
from jax import lax
from jax.experimental import pallas as pl
from jax.experimental.pallas import tpu as pltpu

_ANCHORS = (1.08, 1.19, 3.42, 4.41, 6.63, 11.38, 9.42, 5.11, 16.62, 10.52)
_NA = 5
_NC = 8
_NH = 32
_NW = 32
_NT = 50
_OBJ = 10.0
_THRESH = 0.6
_BIG = 1.0e30


def _loss_body(o_ref, t_ref, a_ref, out_ref):
    f32 = jnp.float32
    i32 = jnp.int32
    o2 = o_ref[0]
    tgt = t_ref[0]

    aw = a_ref[0]
    al = a_ref[1]

    def ch(c):
        return jnp.concatenate(
            [o2[(a * 15 + c) * 8:(a * 15 + c) * 8 + 8] for a in range(_NA)],
            axis=0)

    xs = jax.nn.sigmoid(ch(0))
    ys = jax.nn.sigmoid(ch(1))
    ws = ch(2)
    ls = ch(3)
    ims = ch(4)
    res = ch(5)
    confs = jax.nn.sigmoid(ch(6))

    r_iota = lax.broadcasted_iota(i32, (_NA * 8, 128), 0)
    l_iota = lax.broadcasted_iota(i32, (_NA * 8, 128), 1)
    grid_x = (l_iota & 31).astype(f32)
    grid_y = ((r_iota & 7) * 4 + (l_iota >> 5)).astype(f32)
    a_row = r_iota >> 3
    aw_bc = jnp.zeros((_NA * 8, 128), f32)
    al_bc = jnp.zeros((_NA * 8, 128), f32)
    for a in range(_NA):
        aw_bc = jnp.where(a_row == a, _ANCHORS[2 * a], aw_bc)
        al_bc = jnp.where(a_row == a, _ANCHORS[2 * a + 1], al_bc)

    px = xs + grid_x
    py = ys + grid_y
    pw = jnp.exp(ws) * aw_bc
    plh = jnp.exp(ls) * al_bc
    xl = px - pw * 0.5
    xr = px + pw * 0.5
    yl = py - plh * 0.5
    yr = py + plh * 0.5
    c_t = _THRESH / (1.0 + _THRESH)
    cpa = c_t * (pw * plh)

    gx = tgt[1] * _NW
    gy = tgt[2] * _NH
    gw = tgt[3] * _NW
    gl = tgt[4] * _NH
    gxl = gx - gw * 0.5
    gxr = gx + gw * 0.5
    gyl = gy - gl * 0.5
    gyr = gy + gl * 0.5
    garea = gw * gl

    tt = lax.broadcasted_iota(i32, (_NT, _NT), 0)
    ss = lax.broadcasted_iota(i32, (_NT, _NT), 1)
    zero_seen = jnp.any((ss <= tt) & (tgt[1] == 0.0)[None, :], axis=1)
    valid = jnp.logical_not(zero_seen)
    cga = jnp.where(valid, c_t * garea, _BIG)

    m = jnp.full((_NA * 8, 128), -_BIG, f32)
    for t in range(_NT):
        cw = jnp.minimum(xr, gxr[t]) - jnp.maximum(xl, gxl[t])
        chh = jnp.minimum(yr, gyr[t]) - jnp.maximum(yl, gyl[t])
        inter = jnp.maximum(cw, 0.0) * jnp.maximum(chh, 0.0)
        m = jnp.maximum(m, inter - cga[t])
    noobj = (m <= cpa).astype(f32)
    dense_conf = 0.5 * jnp.sum(confs * confs * noobj)

    inter_a = (jnp.minimum(aw[:, None], gw[None, :])
               * jnp.minimum(al[:, None], gl[None, :]))
    union_a = (aw * al)[:, None] + garea[None, :] - inter_a
    iou_a = inter_a / union_a
    best = jnp.max(iou_a, axis=0)
    a_iota = lax.broadcasted_iota(i32, (_NA, _NT), 0)
    bn = jnp.min(jnp.where(iou_a == best[None, :], a_iota, _NA + 1), axis=0)
    do = valid & (best > 0.0)

    gi = gx.astype(i32)
    gj = gy.astype(i32)
    cellp = gj * _NW + gi
    slot = bn * (_NH * _NW) + cellp

    later = ss > tt
    same = slot[None, :] == slot[:, None]
    clobbered = jnp.any(later & same & do[None, :], axis=1)
    win = do & jnp.logical_not(clobbered)

    vmat = jnp.concatenate(
        [xs, ys, ws, ls, ims, res, confs, noobj,
         ch(7), ch(8), ch(9), ch(10), ch(11), ch(12), ch(13), ch(14)],
        axis=0)
    lane = cellp & 127
    subl = cellp >> 7
    l_oh_iota = lax.broadcasted_iota(i32, (_NT, 128), 1)
    onehot_l = (l_oh_iota == lane[:, None]).astype(f32)
    g640 = lax.dot_general(vmat, onehot_l, (((1,), (1,)), ((), ())),
                           precision=lax.Precision.HIGHEST)
    row_sel = bn * 8 + subl
    r40_iota = lax.broadcasted_iota(i32, (_NA * 8, _NT), 0)
    mask40 = (r40_iota == row_sel[None, :]).astype(f32)
    g = jnp.sum(g640.reshape(16, _NA * 8, _NT) * mask40[None], axis=1)

    xg, yg, wg, lg, img, reg, cg, noobjg = (g[0], g[1], g[2], g[3],
                                            g[4], g[5], g[6], g[7])
    cls_g = g[8:16]

    a_oh = (a_iota == bn[None, :]).astype(f32)
    awb = jnp.sum(a_oh * aw[:, None], axis=0)
    alb = jnp.sum(a_oh * al[:, None], axis=0)

    gif = gi.astype(f32)
    gjf = gj.astype(f32)
    tx = gx - gif
    ty = gy - gjf
    gw_s = jnp.where(do, gw, 1.0)
    gl_s = jnp.where(do, gl, 1.0)
    tw = jnp.log(gw_s / awb)
    tl = jnp.log(gl_s / alb)
    tim = tgt[5]
    tre = tgt[6]

    coord = ((xg - tx) ** 2 + (yg - ty) ** 2 + (wg - tw) ** 2
             + (lg - tl) ** 2 + (img - tim) ** 2 + (reg - tre) ** 2)

    pxg = xg + gif
    pyg = yg + gjf
    pwg = jnp.exp(wg) * awb
    plg = jnp.exp(lg) * alb
    cw2 = jnp.minimum(gxr, pxg + pwg * 0.5) - jnp.maximum(gxl, pxg - pwg * 0.5)
    ch2 = jnp.minimum(gyr, pyg + plg * 0.5) - jnp.maximum(gyl, pyg - plg * 0.5)
    ca2 = cw2 * ch2
    confv = jnp.where((cw2 <= 0.0) | (ch2 <= 0.0), 0.0,
                      ca2 / (garea + pwg * plg - ca2))

    cmax = jnp.max(cls_g, axis=0)
    lse = cmax + jnp.log(jnp.sum(jnp.exp(cls_g - cmax[None]), axis=0))
    c_iota = lax.broadcasted_iota(i32, (_NC, _NT), 0)
    tcls = tgt[0].astype(i32)
    picked = jnp.sum(jnp.where(c_iota == tcls[None, :], cls_g, 0.0), axis=0)

    per_t = (0.5 * coord
             + 0.5 * _OBJ * _OBJ * (cg - confv) ** 2
             - 0.5 * noobjg * cg * cg
             + (lse - picked))
    sparse_loss = jnp.sum(jnp.where(win, per_t, 0.0))

    out_ref[:, :, :] = (dense_conf + sparse_loss)[None, None, None]


def kernel(output, target):
    nB = output.shape[0]
    o = output.reshape(nB, _NA * (7 + _NC) * 8, 128)
    t = target.transpose(0, 2, 1)
    anc = jnp.asarray(_ANCHORS, jnp.float32).reshape(_NA, 2).T
    res = pl.pallas_call(
        _loss_body,
        grid=(nB,),
        in_specs=[
            pl.BlockSpec((1, _NA * (7 + _NC) * 8, 128), lambda b: (b, 0, 0)),
            pl.BlockSpec((1, 7, _NT), lambda b: (b, 0, 0)),
            pl.BlockSpec((2, _NA), lambda b: (0, 0)),
        ],
        out_specs=pl.BlockSpec((1, 1, 1), lambda b: (b, 0, 0)),
        out_shape=jax.ShapeDtypeStruct((nB, 1, 1), jnp.float32),
        compiler_params=pltpu.CompilerParams(
            dimension_semantics=("arbitrary",)),
    )(o, t, anc)
    return jnp.sum(res)


# speedup vs baseline: 1.9103x; 1.9103x over previous
import jax
import jax.numpy as jnp
from jax import lax
from jax.experimental import pallas as pl
from jax.experimental.pallas import tpu as pltpu

_ANCHORS = (1.08, 1.19, 3.42, 4.41, 6.63, 11.38, 9.42, 5.11, 16.62, 10.52)
_NA = 5
_NC = 8
_NH = 32
_NW = 32
_NT = 50
_OBJ = 10.0
_THRESH = 0.6
_CHUNK = 10


def _loss_body(o_ref, t_ref, a_ref, out_ref):
    f32 = jnp.float32
    i32 = jnp.int32
    o = o_ref[0]
    tgt = t_ref[0]

    aw = a_ref[0]
    al = a_ref[1]

    o3 = o.reshape(_NA, 7 + _NC, _NH * _NW)
    xs = jax.nn.sigmoid(o3[:, 0, :])
    ys = jax.nn.sigmoid(o3[:, 1, :])
    ws = o3[:, 2, :]
    ls = o3[:, 3, :]
    ims = o3[:, 4, :]
    res = o3[:, 5, :]
    confs = jax.nn.sigmoid(o3[:, 6, :])

    p = lax.broadcasted_iota(i32, (_NA, _NH * _NW), 1)
    grid_x = (p & (_NW - 1)).astype(f32)
    grid_y = (p >> 5).astype(f32)
    px = xs + grid_x
    py = ys + grid_y
    pw = jnp.exp(ws) * aw[:, None]
    plh = jnp.exp(ls) * al[:, None]
    xl = px - pw * 0.5
    xr = px + pw * 0.5
    yl = py - plh * 0.5
    yr = py + plh * 0.5
    parea = pw * plh

    gx = tgt[1] * _NW
    gy = tgt[2] * _NH
    gw = tgt[3] * _NW
    gl = tgt[4] * _NH
    gxl = gx - gw * 0.5
    gxr = gx + gw * 0.5
    gyl = gy - gl * 0.5
    gyr = gy + gl * 0.5
    garea = gw * gl

    tt = lax.broadcasted_iota(i32, (_NT, _NT), 0)
    ss = lax.broadcasted_iota(i32, (_NT, _NT), 1)
    zero_seen = jnp.any((ss <= tt) & (tgt[1] == 0.0)[None, :], axis=1)
    valid = jnp.logical_not(zero_seen)

    c_t = _THRESH / (1.0 + _THRESH)
    cga = jnp.where(valid, c_t * garea, 1.0e30)
    m = jnp.full((_NA, _NH * _NW), -1.0e30, f32)
    for c0 in range(0, _NT, _CHUNK):
        sl = slice(c0, c0 + _CHUNK)
        cw = (jnp.minimum(xr[None], gxr[sl, None, None])
              - jnp.maximum(xl[None], gxl[sl, None, None]))
        ch = (jnp.minimum(yr[None], gyr[sl, None, None])
              - jnp.maximum(yl[None], gyl[sl, None, None]))
        inter = jnp.maximum(cw, 0.0) * jnp.maximum(ch, 0.0)
        m = jnp.maximum(m, jnp.max(inter - cga[sl, None, None], axis=0))
    noobj = (m <= c_t * parea).astype(f32)
    dense_conf = 0.5 * jnp.sum(confs * confs * noobj)

    inter_a = (jnp.minimum(aw[:, None], gw[None, :])
               * jnp.minimum(al[:, None], gl[None, :]))
    union_a = (aw * al)[:, None] + garea[None, :] - inter_a
    iou_a = inter_a / union_a
    best = jnp.max(iou_a, axis=0)
    a_iota = lax.broadcasted_iota(i32, (_NA, _NT), 0)
    bn = jnp.min(jnp.where(iou_a == best[None, :], a_iota, _NA + 1), axis=0)
    do = valid & (best > 0.0)

    gi = gx.astype(i32)
    gj = gy.astype(i32)
    cellp = gj * _NW + gi
    slot = bn * (_NH * _NW) + cellp

    later = ss > tt
    same = slot[None, :] == slot[:, None]
    clobbered = jnp.any(later & same & do[None, :], axis=1)
    win = do & jnp.logical_not(clobbered)

    vmat = jnp.concatenate(
        [xs, ys, ws, ls, ims, res, confs, noobj,
         o3[:, 7, :], o3[:, 8, :], o3[:, 9, :], o3[:, 10, :],
         o3[:, 11, :], o3[:, 12, :], o3[:, 13, :], o3[:, 14, :]],
        axis=0)
    p_iota = lax.broadcasted_iota(i32, (_NT, _NH * _NW), 1)
    onehot_p = (p_iota == cellp[:, None]).astype(f32)
    g80 = lax.dot_general(vmat, onehot_p, (((1,), (1,)), ((), ())),
                          precision=lax.Precision.HIGHEST)
    a_onehot = (a_iota == bn[None, :]).astype(f32)
    g = jnp.sum(g80.reshape(16, _NA, _NT) * a_onehot[None], axis=1)

    xg, yg, wg, lg, img, reg, cg, noobjg = (g[0], g[1], g[2], g[3],
                                            g[4], g[5], g[6], g[7])
    cls_g = g[8:16]

    awb = jnp.sum(a_onehot * aw[:, None], axis=0)
    alb = jnp.sum(a_onehot * al[:, None], axis=0)

    gif = gi.astype(f32)
    gjf = gj.astype(f32)
    tx = gx - gif
    ty = gy - gjf
    gw_s = jnp.where(do, gw, 1.0)
    gl_s = jnp.where(do, gl, 1.0)
    tw = jnp.log(gw_s / awb)
    tl = jnp.log(gl_s / alb)
    tim = tgt[5]
    tre = tgt[6]

    coord = ((xg - tx) ** 2 + (yg - ty) ** 2 + (wg - tw) ** 2
             + (lg - tl) ** 2 + (img - tim) ** 2 + (reg - tre) ** 2)

    pxg = xg + gif
    pyg = yg + gjf
    pwg = jnp.exp(wg) * awb
    plg = jnp.exp(lg) * alb
    cw2 = jnp.minimum(gxr, pxg + pwg * 0.5) - jnp.maximum(gxl, pxg - pwg * 0.5)
    ch2 = jnp.minimum(gyr, pyg + plg * 0.5) - jnp.maximum(gyl, pyg - plg * 0.5)
    ca2 = cw2 * ch2
    confv = jnp.where((cw2 <= 0.0) | (ch2 <= 0.0), 0.0,
                      ca2 / (garea + pwg * plg - ca2))

    cmax = jnp.max(cls_g, axis=0)
    lse = cmax + jnp.log(jnp.sum(jnp.exp(cls_g - cmax[None]), axis=0))
    c_iota = lax.broadcasted_iota(i32, (_NC, _NT), 0)
    tcls = tgt[0].astype(i32)
    picked = jnp.sum(jnp.where(c_iota == tcls[None, :], cls_g, 0.0), axis=0)

    per_t = (0.5 * coord
             + 0.5 * _OBJ * _OBJ * (cg - confv) ** 2
             - 0.5 * noobjg * cg * cg
             + (lse - picked))
    sparse_loss = jnp.sum(jnp.where(win, per_t, 0.0))

    out_ref[:, :, :] = (dense_conf + sparse_loss)[None, None, None]


def kernel(output, target):
    nB = output.shape[0]
    o = output.reshape(nB, _NA * (7 + _NC), _NH * _NW)
    t = target.transpose(0, 2, 1)
    anc = jnp.asarray(_ANCHORS, jnp.float32).reshape(_NA, 2).T
    res = pl.pallas_call(
        _loss_body,
        grid=(nB,),
        in_specs=[
            pl.BlockSpec((1, _NA * (7 + _NC), _NH * _NW), lambda b: (b, 0, 0)),
            pl.BlockSpec((1, 7, _NT), lambda b: (b, 0, 0)),
            pl.BlockSpec((2, _NA), lambda b: (0, 0)),
        ],
        out_specs=pl.BlockSpec((1, 1, 1), lambda b: (b, 0, 0)),
        out_shape=jax.ShapeDtypeStruct((nB, 1, 1), jnp.float32),
    )(o, t, anc)
    return jnp.sum(res)
